# baseline (speedup 1.0000x reference)
import functools

import jax
import jax.numpy as jnp
from jax import lax
from jax.experimental import pallas as pl
from jax.experimental.pallas import tpu as pltpu
from jax.experimental.pallas import tpu_sc as plsc

N_NODES = 10000
D_IN = 128
D_OUT = 128
N_EDGES = 320000

DE = 144
NC = 2
NS = 16
NW = NC * NS
CHUNK = 128
CH_PER_W = 80
E_PER_W = CH_PER_W * CHUNK
E_PAD = NW * E_PER_W
ACC_ROWS = 10240
DUMP_ROW = N_NODES
ROWS_PER_TILE = ACC_ROWS // NS

_sc_mesh = plsc.VectorSubcoreMesh(
    core_axis_name="c", subcore_axis_name="s", num_cores=NC, num_subcores=NS)


@functools.partial(
    pl.kernel,
    out_type=jax.ShapeDtypeStruct((NC, ACC_ROWS, DE), jnp.float32),
    mesh=_sc_mesh,
    compiler_params=pltpu.CompilerParams(use_tc_tiling_on_sc=False),
    scratch_types=[
        pltpu.VMEM((2, CHUNK), jnp.int32),
        pltpu.VMEM((CHUNK, DE), jnp.float32),
        pltpu.VMEM_SHARED((ACC_ROWS, DE), jnp.float32),
    ],
)
def _sc_aggregate(featext_hbm, edges_hbm, zeros_hbm, out_hbm,
                  ib, rows_v, acc_sh):
    c = lax.axis_index("c")
    s = lax.axis_index("s")
    wid = s * NC + c

    pltpu.sync_copy(zeros_hbm, acc_sh.at[pl.ds(s * ROWS_PER_TILE, ROWS_PER_TILE)])
    plsc.subcore_barrier()

    def body(j, carry):
        pltpu.sync_copy(edges_hbm.at[wid, j], ib)
        pltpu.sync_copy(featext_hbm.at[ib.at[0]], rows_v)
        pltpu.sync_copy(rows_v, acc_sh.at[ib.at[1]], add=True)
        return carry

    lax.fori_loop(0, CH_PER_W, body, 0)
    plsc.subcore_barrier()

    pltpu.sync_copy(acc_sh.at[pl.ds(s * ROWS_PER_TILE, ROWS_PER_TILE)],
                    out_hbm.at[c, pl.ds(s * ROWS_PER_TILE, ROWS_PER_TILE)])


def _tc_combine_body(x_ref, p0_ref, p1_ref, ws_ref, wn_ref, b_ref, o_ref):
    x = x_ref[...]
    p = p0_ref[...] + p1_ref[...]
    neigh_sum = p[:, :D_IN]
    deg = p[:, D_IN:D_IN + 1]
    h_neigh = neigh_sum / jnp.maximum(deg, 1.0)
    dn = (((1,), (1,)), ((), ()))
    out = lax.dot_general(x, ws_ref[...], dn, preferred_element_type=jnp.float32)
    out += lax.dot_general(h_neigh, wn_ref[...], dn, preferred_element_type=jnp.float32)
    o_ref[...] = out + b_ref[...]


def _tc_combine(feat, p0, p1, w_self, w_neigh, b2):
    blk = 1000
    grid = N_NODES // blk
    return pl.pallas_call(
        _tc_combine_body,
        grid=(grid,),
        in_specs=[
            pl.BlockSpec((blk, D_IN), lambda i: (i, 0)),
            pl.BlockSpec((blk, DE), lambda i: (i, 0)),
            pl.BlockSpec((blk, DE), lambda i: (i, 0)),
            pl.BlockSpec((D_OUT, D_IN), lambda i: (0, 0)),
            pl.BlockSpec((D_OUT, D_IN), lambda i: (0, 0)),
            pl.BlockSpec((1, D_OUT), lambda i: (0, 0)),
        ],
        out_specs=pl.BlockSpec((blk, D_OUT), lambda i: (i, 0)),
        out_shape=jax.ShapeDtypeStruct((N_NODES, D_OUT), jnp.float32),
    )(feat, p0, p1, w_self, w_neigh, b2)


def kernel(feat, edge_index, W_self, W_neigh, b_neigh):
    ones = jnp.ones((N_NODES, 1), jnp.float32)
    pad_cols = jnp.zeros((N_NODES, DE - D_IN - 1), jnp.float32)
    feat_ext = jnp.concatenate([feat, ones, pad_cols], axis=1)

    n_pad = E_PAD - N_EDGES
    src_p = jnp.concatenate([edge_index[0], jnp.zeros((n_pad,), jnp.int32)])
    dst_p = jnp.concatenate([edge_index[1],
                             jnp.full((n_pad,), DUMP_ROW, jnp.int32)])
    edges_p = jnp.stack([src_p.reshape(NW, CH_PER_W, CHUNK),
                         dst_p.reshape(NW, CH_PER_W, CHUNK)], axis=2)
    zeros_tile = jnp.zeros((ROWS_PER_TILE, DE), jnp.float32)

    partials = _sc_aggregate(feat_ext, edges_p, zeros_tile)
    b2 = b_neigh.reshape(1, D_OUT)
    return _tc_combine(feat, partials[0], partials[1], W_self, W_neigh, b2)


# speedup vs baseline: 1.0318x; 1.0318x over previous
import functools

import jax
import jax.numpy as jnp
from jax import lax
from jax.experimental import pallas as pl
from jax.experimental.pallas import tpu as pltpu
from jax.experimental.pallas import tpu_sc as plsc

N_NODES = 10000
D_IN = 128
D_OUT = 128
N_EDGES = 320000

DE = 144
NC = 2
NS = 16
NW = NC * NS
CHUNK = 256
CH_PER_W = 40
E_PER_W = CH_PER_W * CHUNK
E_PAD = NW * E_PER_W
ACC_ROWS = 10240
DUMP_ROW = N_NODES
ROWS_PER_TILE = ACC_ROWS // NS

_sc_mesh = plsc.VectorSubcoreMesh(
    core_axis_name="c", subcore_axis_name="s", num_cores=NC, num_subcores=NS)


@functools.partial(
    pl.kernel,
    out_type=jax.ShapeDtypeStruct((NC, ACC_ROWS, DE), jnp.float32),
    mesh=_sc_mesh,
    compiler_params=pltpu.CompilerParams(use_tc_tiling_on_sc=False),
    scratch_types=[
        pltpu.VMEM((CHUNK,), jnp.int32),
        pltpu.VMEM((CHUNK,), jnp.int32),
        pltpu.VMEM((CHUNK, DE), jnp.float32),
        pltpu.VMEM_SHARED((ACC_ROWS, DE), jnp.float32),
    ],
)
def _sc_aggregate(featext_hbm, src_hbm, dst_hbm, zeros_hbm, out_hbm,
                  src_v, dst_v, rows_v, acc_sh):
    c = lax.axis_index("c")
    s = lax.axis_index("s")
    wid = s * NC + c

    pltpu.sync_copy(zeros_hbm, acc_sh.at[pl.ds(s * ROWS_PER_TILE, ROWS_PER_TILE)])
    plsc.subcore_barrier()

    def body(j, carry):
        base = wid * E_PER_W + j * CHUNK
        pltpu.sync_copy(src_hbm.at[pl.ds(base, CHUNK)], src_v)
        pltpu.sync_copy(dst_hbm.at[pl.ds(base, CHUNK)], dst_v)
        pltpu.sync_copy(featext_hbm.at[src_v], rows_v)
        pltpu.sync_copy(rows_v, acc_sh.at[dst_v], add=True)
        return carry

    lax.fori_loop(0, CH_PER_W, body, 0)
    plsc.subcore_barrier()

    pltpu.sync_copy(acc_sh.at[pl.ds(s * ROWS_PER_TILE, ROWS_PER_TILE)],
                    out_hbm.at[c, pl.ds(s * ROWS_PER_TILE, ROWS_PER_TILE)])


def _tc_combine_body(x_ref, p0_ref, p1_ref, ws_ref, wn_ref, b_ref, o_ref):
    x = x_ref[...]
    p = p0_ref[...] + p1_ref[...]
    neigh_sum = p[:, :D_IN]
    deg = p[:, D_IN:D_IN + 1]
    h_neigh = neigh_sum / jnp.maximum(deg, 1.0)
    dn = (((1,), (1,)), ((), ()))
    out = lax.dot_general(x, ws_ref[...], dn, preferred_element_type=jnp.float32)
    out += lax.dot_general(h_neigh, wn_ref[...], dn, preferred_element_type=jnp.float32)
    o_ref[...] = out + b_ref[...]


def _tc_combine(feat, p0, p1, w_self, w_neigh, b2):
    blk = 1000
    grid = N_NODES // blk
    return pl.pallas_call(
        _tc_combine_body,
        grid=(grid,),
        in_specs=[
            pl.BlockSpec((blk, D_IN), lambda i: (i, 0)),
            pl.BlockSpec((blk, DE), lambda i: (i, 0)),
            pl.BlockSpec((blk, DE), lambda i: (i, 0)),
            pl.BlockSpec((D_OUT, D_IN), lambda i: (0, 0)),
            pl.BlockSpec((D_OUT, D_IN), lambda i: (0, 0)),
            pl.BlockSpec((1, D_OUT), lambda i: (0, 0)),
        ],
        out_specs=pl.BlockSpec((blk, D_OUT), lambda i: (i, 0)),
        out_shape=jax.ShapeDtypeStruct((N_NODES, D_OUT), jnp.float32),
    )(feat, p0, p1, w_self, w_neigh, b2)


def kernel(feat, edge_index, W_self, W_neigh, b_neigh):
    ones = jnp.ones((N_NODES, 1), jnp.float32)
    pad_cols = jnp.zeros((N_NODES, DE - D_IN - 1), jnp.float32)
    feat_ext = jnp.concatenate([feat, ones, pad_cols], axis=1)

    n_pad = E_PAD - N_EDGES
    src_p = jnp.concatenate([edge_index[0], jnp.zeros((n_pad,), jnp.int32)])
    dst_p = jnp.concatenate([edge_index[1],
                             jnp.full((n_pad,), DUMP_ROW, jnp.int32)])
    zeros_tile = jnp.zeros((ROWS_PER_TILE, DE), jnp.float32)

    partials = _sc_aggregate(feat_ext, src_p, dst_p, zeros_tile)
    b2 = b_neigh.reshape(1, D_OUT)
    return _tc_combine(feat, partials[0], partials[1], W_self, W_neigh, b2)


# speedup vs baseline: 1.5783x; 1.5296x over previous
import functools

import jax
import jax.numpy as jnp
from jax import lax
from jax.experimental import pallas as pl
from jax.experimental.pallas import tpu as pltpu
from jax.experimental.pallas import tpu_sc as plsc

N_NODES = 10000
D_IN = 128
D_OUT = 128
N_EDGES = 320000

DE = 144
NC = 2
NS = 16
NW = NC * NS
CHUNK = 128
CH_FAST = 117
CH_SLOW = 41
E_PAD = NS * (CH_FAST + CH_SLOW) * CHUNK
ACC_ROWS = 10240
DUMP_ROW = N_NODES
ROWS_PER_TILE = ACC_ROWS // NS

_sc_mesh = plsc.VectorSubcoreMesh(
    core_axis_name="c", subcore_axis_name="s", num_cores=NC, num_subcores=NS)


@functools.partial(
    pl.kernel,
    out_type=jax.ShapeDtypeStruct((NC, ACC_ROWS, DE), jnp.float32),
    mesh=_sc_mesh,
    compiler_params=pltpu.CompilerParams(use_tc_tiling_on_sc=False),
    scratch_types=[
        pltpu.VMEM((CHUNK,), jnp.int32),
        pltpu.VMEM((CHUNK,), jnp.int32),
        pltpu.VMEM((CHUNK, DE), jnp.float32),
        pltpu.VMEM_SHARED((ACC_ROWS, DE), jnp.float32),
    ],
)
def _sc_aggregate(featext_hbm, src_hbm, dst_hbm, zeros_hbm, out_hbm,
                  src_v, dst_v, rows_v, acc_sh):
    c = lax.axis_index("c")
    s = lax.axis_index("s")
    n_chunks = jnp.where(c == 0, CH_FAST, CH_SLOW)
    chunk0 = jnp.where(c == 0, s * CH_FAST, NS * CH_FAST + s * CH_SLOW)

    pltpu.sync_copy(zeros_hbm, acc_sh.at[pl.ds(s * ROWS_PER_TILE, ROWS_PER_TILE)])
    plsc.subcore_barrier()

    def body(j, carry):
        base = (chunk0 + j) * CHUNK
        pltpu.sync_copy(src_hbm.at[pl.ds(base, CHUNK)], src_v)
        pltpu.sync_copy(dst_hbm.at[pl.ds(base, CHUNK)], dst_v)
        pltpu.sync_copy(featext_hbm.at[src_v], rows_v)
        pltpu.sync_copy(rows_v, acc_sh.at[dst_v], add=True)
        return carry

    lax.fori_loop(0, n_chunks, body, 0)
    plsc.subcore_barrier()

    pltpu.sync_copy(acc_sh.at[pl.ds(s * ROWS_PER_TILE, ROWS_PER_TILE)],
                    out_hbm.at[c, pl.ds(s * ROWS_PER_TILE, ROWS_PER_TILE)])


def _tc_combine_body(x_ref, p0_ref, p1_ref, ws_ref, wn_ref, b_ref, o_ref):
    x = x_ref[...]
    p = p0_ref[...] + p1_ref[...]
    neigh_sum = p[:, :D_IN]
    deg = p[:, D_IN:D_IN + 1]
    h_neigh = neigh_sum / jnp.maximum(deg, 1.0)
    dn = (((1,), (1,)), ((), ()))
    out = lax.dot_general(x, ws_ref[...], dn, preferred_element_type=jnp.float32)
    out += lax.dot_general(h_neigh, wn_ref[...], dn, preferred_element_type=jnp.float32)
    o_ref[...] = out + b_ref[...]


def _tc_combine(feat, p0, p1, w_self, w_neigh, b2):
    blk = 1000
    grid = N_NODES // blk
    return pl.pallas_call(
        _tc_combine_body,
        grid=(grid,),
        in_specs=[
            pl.BlockSpec((blk, D_IN), lambda i: (i, 0)),
            pl.BlockSpec((blk, DE), lambda i: (i, 0)),
            pl.BlockSpec((blk, DE), lambda i: (i, 0)),
            pl.BlockSpec((D_OUT, D_IN), lambda i: (0, 0)),
            pl.BlockSpec((D_OUT, D_IN), lambda i: (0, 0)),
            pl.BlockSpec((1, D_OUT), lambda i: (0, 0)),
        ],
        out_specs=pl.BlockSpec((blk, D_OUT), lambda i: (i, 0)),
        out_shape=jax.ShapeDtypeStruct((N_NODES, D_OUT), jnp.float32),
    )(feat, p0, p1, w_self, w_neigh, b2)


def kernel(feat, edge_index, W_self, W_neigh, b_neigh):
    ones = jnp.ones((N_NODES, 1), jnp.float32)
    pad_cols = jnp.zeros((N_NODES, DE - D_IN - 1), jnp.float32)
    feat_ext = jnp.concatenate([feat, ones, pad_cols], axis=1)

    n_pad = E_PAD - N_EDGES
    src_p = jnp.concatenate([edge_index[0], jnp.zeros((n_pad,), jnp.int32)])
    dst_p = jnp.concatenate([edge_index[1],
                             jnp.full((n_pad,), DUMP_ROW, jnp.int32)])
    zeros_tile = jnp.zeros((ROWS_PER_TILE, DE), jnp.float32)

    partials = _sc_aggregate(feat_ext, src_p, dst_p, zeros_tile)
    b2 = b_neigh.reshape(1, D_OUT)
    return _tc_combine(feat, partials[0], partials[1], W_self, W_neigh, b2)


# speedup vs baseline: 1.6915x; 1.0718x over previous
import functools

import jax
import jax.numpy as jnp
from jax import lax
from jax.experimental import pallas as pl
from jax.experimental.pallas import tpu as pltpu
from jax.experimental.pallas import tpu_sc as plsc

N_NODES = 10000
D_IN = 128
D_OUT = 128
N_EDGES = 320000

DE = 144
NC = 2
NS = 16
NW = NC * NS
CHUNK = 128
N_CHUNKS = N_EDGES // CHUNK
CH_SC0 = 1792
CH_SC1 = N_CHUNKS - CH_SC0
CH1_BASE = CH_SC1 // NS
CH1_XTRA = CH_SC1 % NS
ACC_ROWS = 10240
ROWS_PER_TILE = ACC_ROWS // NS

_sc_mesh = plsc.VectorSubcoreMesh(
    core_axis_name="c", subcore_axis_name="s", num_cores=NC, num_subcores=NS)


@functools.partial(
    pl.kernel,
    out_type=jax.ShapeDtypeStruct((NC, ACC_ROWS, DE), jnp.float32),
    mesh=_sc_mesh,
    compiler_params=pltpu.CompilerParams(use_tc_tiling_on_sc=False),
    scratch_types=[
        pltpu.VMEM((CHUNK,), jnp.int32),
        pltpu.VMEM((CHUNK,), jnp.int32),
        pltpu.VMEM((CHUNK, DE), jnp.float32),
        pltpu.VMEM_SHARED((ACC_ROWS, DE), jnp.float32),
    ],
)
def _sc_aggregate(featext_hbm, src_hbm, dst_hbm, zeros_hbm, out_hbm,
                  src_v, dst_v, rows_v, acc_sh):
    c = lax.axis_index("c")
    s = lax.axis_index("s")
    n_chunks = jnp.where(c == 0, CH_SC0 // NS, CH1_BASE + (s < CH1_XTRA))
    chunk0 = jnp.where(c == 0, s * (CH_SC0 // NS),
                       CH_SC0 + s * CH1_BASE + jnp.minimum(s, CH1_XTRA))

    pltpu.sync_copy(zeros_hbm, acc_sh.at[pl.ds(s * ROWS_PER_TILE, ROWS_PER_TILE)])
    plsc.subcore_barrier()

    def body(j, carry):
        base = (chunk0 + j) * CHUNK
        pltpu.sync_copy(src_hbm.at[pl.ds(base, CHUNK)], src_v)
        pltpu.sync_copy(dst_hbm.at[pl.ds(base, CHUNK)], dst_v)
        pltpu.sync_copy(featext_hbm.at[src_v], rows_v)
        pltpu.sync_copy(rows_v, acc_sh.at[dst_v], add=True)
        return carry

    lax.fori_loop(0, n_chunks, body, 0)
    plsc.subcore_barrier()

    pltpu.sync_copy(acc_sh.at[pl.ds(s * ROWS_PER_TILE, ROWS_PER_TILE)],
                    out_hbm.at[c, pl.ds(s * ROWS_PER_TILE, ROWS_PER_TILE)])


_DN = (((1,), (1,)), ((), ()))
_BLK = 1000


def _tc_self_body(x_ref, ws_ref, b_ref, o_ref):
    o_ref[...] = lax.dot_general(x_ref[...], ws_ref[...], _DN,
                                 preferred_element_type=jnp.float32) + b_ref[...]


def _tc_self(feat, w_self, b2):
    return pl.pallas_call(
        _tc_self_body,
        grid=(N_NODES // _BLK,),
        in_specs=[
            pl.BlockSpec((_BLK, D_IN), lambda i: (i, 0)),
            pl.BlockSpec((D_OUT, D_IN), lambda i: (0, 0)),
            pl.BlockSpec((1, D_OUT), lambda i: (0, 0)),
        ],
        out_specs=pl.BlockSpec((_BLK, D_OUT), lambda i: (i, 0)),
        out_shape=jax.ShapeDtypeStruct((N_NODES, D_OUT), jnp.float32),
    )(feat, w_self, b2)


def _tc_combine_body(p0_ref, p1_ref, wn_ref, s_ref, o_ref):
    p = p0_ref[...] + p1_ref[...]
    h_neigh = p[:, :D_IN] / jnp.maximum(p[:, D_IN:D_IN + 1], 1.0)
    o_ref[...] = lax.dot_general(h_neigh, wn_ref[...], _DN,
                                 preferred_element_type=jnp.float32) + s_ref[...]


def _tc_combine(p0, p1, w_neigh, self_part):
    return pl.pallas_call(
        _tc_combine_body,
        grid=(N_NODES // _BLK,),
        in_specs=[
            pl.BlockSpec((_BLK, DE), lambda i: (i, 0)),
            pl.BlockSpec((_BLK, DE), lambda i: (i, 0)),
            pl.BlockSpec((D_OUT, D_IN), lambda i: (0, 0)),
            pl.BlockSpec((_BLK, D_OUT), lambda i: (i, 0)),
        ],
        out_specs=pl.BlockSpec((_BLK, D_OUT), lambda i: (i, 0)),
        out_shape=jax.ShapeDtypeStruct((N_NODES, D_OUT), jnp.float32),
    )(p0, p1, w_neigh, self_part)


def kernel(feat, edge_index, W_self, W_neigh, b_neigh):
    ones = jnp.ones((N_NODES, 1), jnp.float32)
    pad_cols = jnp.zeros((N_NODES, DE - D_IN - 1), jnp.float32)
    feat_ext = jnp.concatenate([feat, ones, pad_cols], axis=1)
    zeros_tile = jnp.zeros((ROWS_PER_TILE, DE), jnp.float32)

    b2 = b_neigh.reshape(1, D_OUT)
    self_part = _tc_self(feat, W_self, b2)
    partials = _sc_aggregate(feat_ext, edge_index[0], edge_index[1], zeros_tile)
    return _tc_combine(partials[0], partials[1], W_neigh, self_part)


# speedup vs baseline: 2.7345x; 1.6166x over previous
import functools

import jax
import jax.numpy as jnp
from jax import lax
from jax.experimental import pallas as pl
from jax.experimental.pallas import tpu as pltpu
from jax.experimental.pallas import tpu_sc as plsc

N_NODES = 10000
D_IN = 128
D_OUT = 128
N_EDGES = 320000

DE = 144
NC = 2
NS = 16
NW = NC * NS
CHUNK = 128
N_CHUNKS = N_EDGES // CHUNK
CH_SC0 = 1792
CH_SC1 = N_CHUNKS - CH_SC0
CH1_BASE = CH_SC1 // NS
CH1_XTRA = CH_SC1 % NS
ACC_ROWS = 10240
ROWS_PER_TILE = ACC_ROWS // NS

_sc_mesh = plsc.VectorSubcoreMesh(
    core_axis_name="c", subcore_axis_name="s", num_cores=NC, num_subcores=NS)


@functools.partial(
    pl.kernel,
    out_type=jax.ShapeDtypeStruct((NC, ACC_ROWS, DE), jnp.float32),
    mesh=_sc_mesh,
    compiler_params=pltpu.CompilerParams(use_tc_tiling_on_sc=False),
    scratch_types=[
        pltpu.VMEM((CHUNK,), jnp.int32),
        pltpu.VMEM((CHUNK,), jnp.int32),
        pltpu.VMEM((CHUNK,), jnp.int32),
        pltpu.VMEM((CHUNK,), jnp.int32),
        pltpu.VMEM((CHUNK, DE), jnp.float32),
        pltpu.VMEM((CHUNK, DE), jnp.float32),
        pltpu.SemaphoreType.DMA,
        pltpu.SemaphoreType.DMA,
        pltpu.VMEM_SHARED((ACC_ROWS, DE), jnp.float32),
    ],
)
def _sc_aggregate(featext_hbm, src_hbm, dst_hbm, zeros_hbm, out_hbm,
                  src0, dst0, src1, dst1, rows0, rows1, gsem, isem, acc_sh):
    c = lax.axis_index("c")
    s = lax.axis_index("s")
    n_chunks = jnp.where(c == 0, CH_SC0 // NS, CH1_BASE + 2 * (s < CH1_XTRA // 2))
    chunk0 = jnp.where(c == 0, s * (CH_SC0 // NS),
                       CH_SC0 + s * CH1_BASE + 2 * jnp.minimum(s, CH1_XTRA // 2))

    def load_idx(k, sv, dv):
        base = (chunk0 + k) * CHUNK
        pltpu.async_copy(src_hbm.at[pl.ds(base, CHUNK)], sv, isem)
        pltpu.async_copy(dst_hbm.at[pl.ds(base, CHUNK)], dv, isem)

    def wait_idx(sv, dv):
        pltpu.make_async_copy(src_hbm.at[pl.ds(0, CHUNK)], sv, isem).wait()
        pltpu.make_async_copy(dst_hbm.at[pl.ds(0, CHUNK)], dv, isem).wait()

    def wait_rows(sv, buf):
        pltpu.make_async_copy(featext_hbm.at[sv], buf, gsem).wait()

    pltpu.sync_copy(zeros_hbm, acc_sh.at[pl.ds(s * ROWS_PER_TILE, ROWS_PER_TILE)])
    plsc.subcore_barrier()

    pltpu.sync_copy(src_hbm.at[pl.ds(chunk0 * CHUNK, CHUNK)], src0)
    pltpu.sync_copy(dst_hbm.at[pl.ds(chunk0 * CHUNK, CHUNK)], dst0)
    pltpu.async_copy(featext_hbm.at[src0], rows0, gsem)
    load_idx(1, src1, dst1)

    def body(j2, carry):
        j = 2 * j2
        wait_idx(src1, dst1)
        wait_rows(src0, rows0)
        pltpu.async_copy(featext_hbm.at[src1], rows1, gsem)
        pltpu.sync_copy(rows0, acc_sh.at[dst0], add=True)

        @pl.when(j + 2 < n_chunks)
        def _():
            load_idx(j + 2, src0, dst0)

        wait_rows(src1, rows1)

        @pl.when(j + 2 < n_chunks)
        def _():
            wait_idx(src0, dst0)
            pltpu.async_copy(featext_hbm.at[src0], rows0, gsem)

        pltpu.sync_copy(rows1, acc_sh.at[dst1], add=True)

        @pl.when(j + 3 < n_chunks)
        def _():
            load_idx(j + 3, src1, dst1)

        return carry

    lax.fori_loop(0, n_chunks // 2, body, 0)
    plsc.subcore_barrier()

    pltpu.sync_copy(acc_sh.at[pl.ds(s * ROWS_PER_TILE, ROWS_PER_TILE)],
                    out_hbm.at[c, pl.ds(s * ROWS_PER_TILE, ROWS_PER_TILE)])


_DN = (((1,), (1,)), ((), ()))
_BLK = 1000


def _tc_self_body(x_ref, ws_ref, b_ref, o_ref):
    o_ref[...] = lax.dot_general(x_ref[...], ws_ref[...], _DN,
                                 preferred_element_type=jnp.float32) + b_ref[...]


def _tc_self(feat, w_self, b2):
    return pl.pallas_call(
        _tc_self_body,
        grid=(N_NODES // _BLK,),
        in_specs=[
            pl.BlockSpec((_BLK, D_IN), lambda i: (i, 0)),
            pl.BlockSpec((D_OUT, D_IN), lambda i: (0, 0)),
            pl.BlockSpec((1, D_OUT), lambda i: (0, 0)),
        ],
        out_specs=pl.BlockSpec((_BLK, D_OUT), lambda i: (i, 0)),
        out_shape=jax.ShapeDtypeStruct((N_NODES, D_OUT), jnp.float32),
    )(feat, w_self, b2)


def _tc_combine_body(p0_ref, p1_ref, wn_ref, s_ref, o_ref):
    p = p0_ref[...] + p1_ref[...]
    h_neigh = p[:, :D_IN] / jnp.maximum(p[:, D_IN:D_IN + 1], 1.0)
    o_ref[...] = lax.dot_general(h_neigh, wn_ref[...], _DN,
                                 preferred_element_type=jnp.float32) + s_ref[...]


def _tc_combine(p0, p1, w_neigh, self_part):
    return pl.pallas_call(
        _tc_combine_body,
        grid=(N_NODES // _BLK,),
        in_specs=[
            pl.BlockSpec((_BLK, DE), lambda i: (i, 0)),
            pl.BlockSpec((_BLK, DE), lambda i: (i, 0)),
            pl.BlockSpec((D_OUT, D_IN), lambda i: (0, 0)),
            pl.BlockSpec((_BLK, D_OUT), lambda i: (i, 0)),
        ],
        out_specs=pl.BlockSpec((_BLK, D_OUT), lambda i: (i, 0)),
        out_shape=jax.ShapeDtypeStruct((N_NODES, D_OUT), jnp.float32),
    )(p0, p1, w_neigh, self_part)


def kernel(feat, edge_index, W_self, W_neigh, b_neigh):
    ones = jnp.ones((N_NODES, 1), jnp.float32)
    pad_cols = jnp.zeros((N_NODES, DE - D_IN - 1), jnp.float32)
    feat_ext = jnp.concatenate([feat, ones, pad_cols], axis=1)
    zeros_tile = jnp.zeros((ROWS_PER_TILE, DE), jnp.float32)

    b2 = b_neigh.reshape(1, D_OUT)
    self_part = _tc_self(feat, W_self, b2)
    partials = _sc_aggregate(feat_ext, edge_index[0], edge_index[1], zeros_tile)
    return _tc_combine(partials[0], partials[1], W_neigh, self_part)


# speedup vs baseline: 3.1805x; 1.1631x over previous
import functools

import jax
import jax.numpy as jnp
from jax import lax
from jax.experimental import pallas as pl
from jax.experimental.pallas import tpu as pltpu
from jax.experimental.pallas import tpu_sc as plsc

N_NODES = 10000
D_IN = 128
D_OUT = 128
N_EDGES = 320000

DE = 144
NC = 2
NS = 16
NW = NC * NS
CHUNK = 128
N_CHUNKS = N_EDGES // CHUNK
CH_SC0 = 1344
CH_SC1 = N_CHUNKS - CH_SC0
CH1_BASE = CH_SC1 // NS
CH1_XTRA = CH_SC1 % NS
ACC_ROWS = 10240
ROWS_PER_TILE = ACC_ROWS // NS

_sc_mesh = plsc.VectorSubcoreMesh(
    core_axis_name="c", subcore_axis_name="s", num_cores=NC, num_subcores=NS)


@functools.partial(
    pl.kernel,
    out_type=jax.ShapeDtypeStruct((NC, ACC_ROWS, DE), jnp.float32),
    mesh=_sc_mesh,
    compiler_params=pltpu.CompilerParams(use_tc_tiling_on_sc=False),
    scratch_types=[
        pltpu.VMEM((CHUNK,), jnp.int32),
        pltpu.VMEM((CHUNK,), jnp.int32),
        pltpu.VMEM((CHUNK,), jnp.int32),
        pltpu.VMEM((CHUNK,), jnp.int32),
        pltpu.VMEM((CHUNK, DE), jnp.float32),
        pltpu.VMEM((CHUNK, DE), jnp.float32),
        pltpu.SemaphoreType.DMA,
        pltpu.SemaphoreType.DMA,
        pltpu.VMEM_SHARED((ACC_ROWS, DE), jnp.float32),
    ],
)
def _sc_aggregate(featext_hbm, src_hbm, dst_hbm, zeros_hbm, out_hbm,
                  src0, dst0, src1, dst1, rows0, rows1, gsem, isem, acc_sh):
    c = lax.axis_index("c")
    s = lax.axis_index("s")
    n_chunks = jnp.where(c == 0, CH_SC0 // NS, CH1_BASE + 2 * (s < CH1_XTRA // 2))
    chunk0 = jnp.where(c == 0, s * (CH_SC0 // NS),
                       CH_SC0 + s * CH1_BASE + 2 * jnp.minimum(s, CH1_XTRA // 2))

    def load_idx(k, sv, dv):
        base = (chunk0 + k) * CHUNK
        pltpu.async_copy(src_hbm.at[pl.ds(base, CHUNK)], sv, isem)
        pltpu.async_copy(dst_hbm.at[pl.ds(base, CHUNK)], dv, isem)

    def wait_idx(sv, dv):
        pltpu.make_async_copy(src_hbm.at[pl.ds(0, CHUNK)], sv, isem).wait()
        pltpu.make_async_copy(dst_hbm.at[pl.ds(0, CHUNK)], dv, isem).wait()

    def wait_rows(sv, buf):
        pltpu.make_async_copy(featext_hbm.at[sv], buf, gsem).wait()

    pltpu.sync_copy(zeros_hbm, acc_sh.at[pl.ds(s * ROWS_PER_TILE, ROWS_PER_TILE)])
    plsc.subcore_barrier()

    pltpu.sync_copy(src_hbm.at[pl.ds(chunk0 * CHUNK, CHUNK)], src0)
    pltpu.sync_copy(dst_hbm.at[pl.ds(chunk0 * CHUNK, CHUNK)], dst0)
    pltpu.async_copy(featext_hbm.at[src0], rows0, gsem)
    load_idx(1, src1, dst1)

    def body(j2, carry):
        j = 2 * j2
        wait_idx(src1, dst1)
        wait_rows(src0, rows0)
        pltpu.async_copy(featext_hbm.at[src1], rows1, gsem)
        pltpu.sync_copy(rows0, acc_sh.at[dst0], add=True)

        @pl.when(j + 2 < n_chunks)
        def _():
            load_idx(j + 2, src0, dst0)

        wait_rows(src1, rows1)

        @pl.when(j + 2 < n_chunks)
        def _():
            wait_idx(src0, dst0)
            pltpu.async_copy(featext_hbm.at[src0], rows0, gsem)

        pltpu.sync_copy(rows1, acc_sh.at[dst1], add=True)

        @pl.when(j + 3 < n_chunks)
        def _():
            load_idx(j + 3, src1, dst1)

        return carry

    lax.fori_loop(0, n_chunks // 2, body, 0)
    plsc.subcore_barrier()

    pltpu.sync_copy(acc_sh.at[pl.ds(s * ROWS_PER_TILE, ROWS_PER_TILE)],
                    out_hbm.at[c, pl.ds(s * ROWS_PER_TILE, ROWS_PER_TILE)])


_DN = (((1,), (1,)), ((), ()))
_BLK = 1000


def _tc_self_body(x_ref, ws_ref, b_ref, o_ref):
    o_ref[...] = lax.dot_general(x_ref[...], ws_ref[...], _DN,
                                 preferred_element_type=jnp.float32) + b_ref[...]


def _tc_self(feat, w_self, b2):
    return pl.pallas_call(
        _tc_self_body,
        grid=(N_NODES // _BLK,),
        in_specs=[
            pl.BlockSpec((_BLK, D_IN), lambda i: (i, 0)),
            pl.BlockSpec((D_OUT, D_IN), lambda i: (0, 0)),
            pl.BlockSpec((1, D_OUT), lambda i: (0, 0)),
        ],
        out_specs=pl.BlockSpec((_BLK, D_OUT), lambda i: (i, 0)),
        out_shape=jax.ShapeDtypeStruct((N_NODES, D_OUT), jnp.float32),
    )(feat, w_self, b2)


def _tc_combine_body(p0_ref, p1_ref, wn_ref, s_ref, o_ref):
    p = p0_ref[...] + p1_ref[...]
    h_neigh = p[:, :D_IN] / jnp.maximum(p[:, D_IN:D_IN + 1], 1.0)
    o_ref[...] = lax.dot_general(h_neigh, wn_ref[...], _DN,
                                 preferred_element_type=jnp.float32) + s_ref[...]


def _tc_combine(p0, p1, w_neigh, self_part):
    return pl.pallas_call(
        _tc_combine_body,
        grid=(N_NODES // _BLK,),
        in_specs=[
            pl.BlockSpec((_BLK, DE), lambda i: (i, 0)),
            pl.BlockSpec((_BLK, DE), lambda i: (i, 0)),
            pl.BlockSpec((D_OUT, D_IN), lambda i: (0, 0)),
            pl.BlockSpec((_BLK, D_OUT), lambda i: (i, 0)),
        ],
        out_specs=pl.BlockSpec((_BLK, D_OUT), lambda i: (i, 0)),
        out_shape=jax.ShapeDtypeStruct((N_NODES, D_OUT), jnp.float32),
    )(p0, p1, w_neigh, self_part)


def kernel(feat, edge_index, W_self, W_neigh, b_neigh):
    ones = jnp.ones((N_NODES, 1), jnp.float32)
    pad_cols = jnp.zeros((N_NODES, DE - D_IN - 1), jnp.float32)
    feat_ext = jnp.concatenate([feat, ones, pad_cols], axis=1)
    zeros_tile = jnp.zeros((ROWS_PER_TILE, DE), jnp.float32)

    b2 = b_neigh.reshape(1, D_OUT)
    self_part = _tc_self(feat, W_self, b2)
    partials = _sc_aggregate(feat_ext, edge_index[0], edge_index[1], zeros_tile)
    return _tc_combine(partials[0], partials[1], W_neigh, self_part)


# speedup vs baseline: 3.2719x; 1.0288x over previous
import functools

import jax
import jax.numpy as jnp
from jax import lax
from jax.experimental import pallas as pl
from jax.experimental.pallas import tpu as pltpu
from jax.experimental.pallas import tpu_sc as plsc

N_NODES = 10000
D_IN = 128
D_OUT = 128
N_EDGES = 320000

DE = 144
NC = 2
NS = 16
NW = NC * NS
CHUNK = 128
N_CHUNKS = N_EDGES // CHUNK
CH_SC0 = 1248
CH_SC1 = N_CHUNKS - CH_SC0
CH1_BASE = CH_SC1 // NS
CH1_XTRA = CH_SC1 % NS
ACC_ROWS = 10240
ROWS_PER_TILE = ACC_ROWS // NS

_sc_mesh = plsc.VectorSubcoreMesh(
    core_axis_name="c", subcore_axis_name="s", num_cores=NC, num_subcores=NS)


@functools.partial(
    pl.kernel,
    out_type=jax.ShapeDtypeStruct((NC, ACC_ROWS, DE), jnp.float32),
    mesh=_sc_mesh,
    compiler_params=pltpu.CompilerParams(use_tc_tiling_on_sc=False),
    scratch_types=[
        pltpu.VMEM((CHUNK,), jnp.int32),
        pltpu.VMEM((CHUNK,), jnp.int32),
        pltpu.VMEM((CHUNK,), jnp.int32),
        pltpu.VMEM((CHUNK,), jnp.int32),
        pltpu.VMEM((CHUNK, DE), jnp.float32),
        pltpu.VMEM((CHUNK, DE), jnp.float32),
        pltpu.SemaphoreType.DMA,
        pltpu.SemaphoreType.DMA,
        pltpu.VMEM_SHARED((ACC_ROWS, DE), jnp.float32),
    ],
)
def _sc_aggregate(featext_hbm, src_hbm, dst_hbm, zeros_hbm, out_hbm,
                  src0, dst0, src1, dst1, rows0, rows1, gsem, isem, acc_sh):
    c = lax.axis_index("c")
    s = lax.axis_index("s")
    n_chunks = jnp.where(c == 0, CH_SC0 // NS, CH1_BASE + 2 * (s < CH1_XTRA // 2))
    chunk0 = jnp.where(c == 0, s * (CH_SC0 // NS),
                       CH_SC0 + s * CH1_BASE + 2 * jnp.minimum(s, CH1_XTRA // 2))

    def load_idx(k, sv, dv):
        base = (chunk0 + k) * CHUNK
        pltpu.async_copy(src_hbm.at[pl.ds(base, CHUNK)], sv, isem)
        pltpu.async_copy(dst_hbm.at[pl.ds(base, CHUNK)], dv, isem)

    def wait_idx(sv, dv):
        pltpu.make_async_copy(src_hbm.at[pl.ds(0, CHUNK)], sv, isem).wait()
        pltpu.make_async_copy(dst_hbm.at[pl.ds(0, CHUNK)], dv, isem).wait()

    def wait_rows(sv, buf):
        pltpu.make_async_copy(featext_hbm.at[sv], buf, gsem).wait()

    pltpu.sync_copy(zeros_hbm, acc_sh.at[pl.ds(s * ROWS_PER_TILE, ROWS_PER_TILE)])
    plsc.subcore_barrier()

    pltpu.sync_copy(src_hbm.at[pl.ds(chunk0 * CHUNK, CHUNK)], src0)
    pltpu.sync_copy(dst_hbm.at[pl.ds(chunk0 * CHUNK, CHUNK)], dst0)
    pltpu.async_copy(featext_hbm.at[src0], rows0, gsem)
    load_idx(1, src1, dst1)

    def body(j2, carry):
        j = 2 * j2
        wait_idx(src1, dst1)
        wait_rows(src0, rows0)
        pltpu.async_copy(featext_hbm.at[src1], rows1, gsem)
        pltpu.sync_copy(rows0, acc_sh.at[dst0], add=True)

        @pl.when(j + 2 < n_chunks)
        def _():
            load_idx(j + 2, src0, dst0)

        wait_rows(src1, rows1)

        @pl.when(j + 2 < n_chunks)
        def _():
            wait_idx(src0, dst0)
            pltpu.async_copy(featext_hbm.at[src0], rows0, gsem)

        pltpu.sync_copy(rows1, acc_sh.at[dst1], add=True)

        @pl.when(j + 3 < n_chunks)
        def _():
            load_idx(j + 3, src1, dst1)

        return carry

    lax.fori_loop(0, n_chunks // 2, body, 0)
    plsc.subcore_barrier()

    pltpu.sync_copy(acc_sh.at[pl.ds(s * ROWS_PER_TILE, ROWS_PER_TILE)],
                    out_hbm.at[c, pl.ds(s * ROWS_PER_TILE, ROWS_PER_TILE)])


_DN = (((1,), (1,)), ((), ()))
_BLK = 1000


def _tc_self_body(x_ref, ws_ref, b_ref, o_ref):
    o_ref[...] = lax.dot_general(x_ref[...], ws_ref[...], _DN,
                                 preferred_element_type=jnp.float32) + b_ref[...]


def _tc_self(feat, w_self, b2):
    return pl.pallas_call(
        _tc_self_body,
        grid=(N_NODES // _BLK,),
        in_specs=[
            pl.BlockSpec((_BLK, D_IN), lambda i: (i, 0)),
            pl.BlockSpec((D_OUT, D_IN), lambda i: (0, 0)),
            pl.BlockSpec((1, D_OUT), lambda i: (0, 0)),
        ],
        out_specs=pl.BlockSpec((_BLK, D_OUT), lambda i: (i, 0)),
        out_shape=jax.ShapeDtypeStruct((N_NODES, D_OUT), jnp.float32),
    )(feat, w_self, b2)


def _tc_combine_body(p0_ref, p1_ref, wn_ref, s_ref, o_ref):
    p = p0_ref[...] + p1_ref[...]
    h_neigh = p[:, :D_IN] / jnp.maximum(p[:, D_IN:D_IN + 1], 1.0)
    o_ref[...] = lax.dot_general(h_neigh, wn_ref[...], _DN,
                                 preferred_element_type=jnp.float32) + s_ref[...]


def _tc_combine(p0, p1, w_neigh, self_part):
    return pl.pallas_call(
        _tc_combine_body,
        grid=(N_NODES // _BLK,),
        in_specs=[
            pl.BlockSpec((_BLK, DE), lambda i: (i, 0)),
            pl.BlockSpec((_BLK, DE), lambda i: (i, 0)),
            pl.BlockSpec((D_OUT, D_IN), lambda i: (0, 0)),
            pl.BlockSpec((_BLK, D_OUT), lambda i: (i, 0)),
        ],
        out_specs=pl.BlockSpec((_BLK, D_OUT), lambda i: (i, 0)),
        out_shape=jax.ShapeDtypeStruct((N_NODES, D_OUT), jnp.float32),
    )(p0, p1, w_neigh, self_part)


def kernel(feat, edge_index, W_self, W_neigh, b_neigh):
    ones = jnp.ones((N_NODES, 1), jnp.float32)
    pad_cols = jnp.zeros((N_NODES, DE - D_IN - 1), jnp.float32)
    feat_ext = jnp.concatenate([feat, ones, pad_cols], axis=1)
    zeros_tile = jnp.zeros((ROWS_PER_TILE, DE), jnp.float32)

    b2 = b_neigh.reshape(1, D_OUT)
    self_part = _tc_self(feat, W_self, b2)
    partials = _sc_aggregate(feat_ext, edge_index[0], edge_index[1], zeros_tile)
    return _tc_combine(partials[0], partials[1], W_neigh, self_part)


# speedup vs baseline: 3.4406x; 1.0516x over previous
import functools

import jax
import jax.numpy as jnp
from jax import lax
from jax.experimental import pallas as pl
from jax.experimental.pallas import tpu as pltpu
from jax.experimental.pallas import tpu_sc as plsc

N_NODES = 10000
D_IN = 128
D_OUT = 128
N_EDGES = 320000

DE = 144
NC = 2
NS = 16
NW = NC * NS
CHUNK = 128
N_CHUNKS = N_EDGES // CHUNK
CH_SC0 = 1248
CH_SC1 = N_CHUNKS - CH_SC0
CH1_BASE = CH_SC1 // NS
CH1_XTRA = CH_SC1 % NS
ACC_ROWS = 10240
ROWS_PER_TILE = ACC_ROWS // NS

_sc_mesh = plsc.VectorSubcoreMesh(
    core_axis_name="c", subcore_axis_name="s", num_cores=NC, num_subcores=NS)


@functools.partial(
    pl.kernel,
    out_type=jax.ShapeDtypeStruct((NC, ACC_ROWS, DE), jnp.float32),
    mesh=_sc_mesh,
    compiler_params=pltpu.CompilerParams(use_tc_tiling_on_sc=False),
    scratch_types=[
        pltpu.VMEM((CHUNK,), jnp.int32),
        pltpu.VMEM((CHUNK,), jnp.int32),
        pltpu.VMEM((CHUNK,), jnp.int32),
        pltpu.VMEM((CHUNK,), jnp.int32),
        pltpu.VMEM((CHUNK, DE), jnp.float32),
        pltpu.VMEM((CHUNK, DE), jnp.float32),
        pltpu.SemaphoreType.DMA,
        pltpu.SemaphoreType.DMA,
        pltpu.VMEM_SHARED((ACC_ROWS, DE), jnp.float32),
    ],
)
def _sc_aggregate(featext_hbm, edges_hbm, zeros_hbm, out_hbm,
                  src0, dst0, src1, dst1, rows0, rows1, gsem, isem, acc_sh):
    c = lax.axis_index("c")
    s = lax.axis_index("s")
    n_chunks = jnp.where(c == 0, CH_SC0 // NS, CH1_BASE + 2 * (s < CH1_XTRA // 2))
    chunk0 = jnp.where(c == 0, s * (CH_SC0 // NS),
                       CH_SC0 + s * CH1_BASE + 2 * jnp.minimum(s, CH1_XTRA // 2))

    def load_idx(k, sv, dv):
        base = (chunk0 + k) * CHUNK
        pltpu.async_copy(edges_hbm.at[0, pl.ds(base, CHUNK)], sv, isem)
        pltpu.async_copy(edges_hbm.at[1, pl.ds(base, CHUNK)], dv, isem)

    def wait_idx(sv, dv):
        pltpu.make_async_copy(edges_hbm.at[0, pl.ds(0, CHUNK)], sv, isem).wait()
        pltpu.make_async_copy(edges_hbm.at[1, pl.ds(0, CHUNK)], dv, isem).wait()

    def wait_rows(sv, buf):
        pltpu.make_async_copy(featext_hbm.at[sv], buf, gsem).wait()

    pltpu.sync_copy(zeros_hbm, acc_sh.at[pl.ds(s * ROWS_PER_TILE, ROWS_PER_TILE)])
    plsc.subcore_barrier()

    pltpu.sync_copy(edges_hbm.at[0, pl.ds(chunk0 * CHUNK, CHUNK)], src0)
    pltpu.sync_copy(edges_hbm.at[1, pl.ds(chunk0 * CHUNK, CHUNK)], dst0)
    pltpu.async_copy(featext_hbm.at[src0], rows0, gsem)
    load_idx(1, src1, dst1)

    def body(j2, carry):
        j = 2 * j2
        wait_idx(src1, dst1)
        wait_rows(src0, rows0)
        pltpu.async_copy(featext_hbm.at[src1], rows1, gsem)
        pltpu.sync_copy(rows0, acc_sh.at[dst0], add=True)

        @pl.when(j + 2 < n_chunks)
        def _():
            load_idx(j + 2, src0, dst0)

        wait_rows(src1, rows1)

        @pl.when(j + 2 < n_chunks)
        def _():
            wait_idx(src0, dst0)
            pltpu.async_copy(featext_hbm.at[src0], rows0, gsem)

        pltpu.sync_copy(rows1, acc_sh.at[dst1], add=True)

        @pl.when(j + 3 < n_chunks)
        def _():
            load_idx(j + 3, src1, dst1)

        return carry

    lax.fori_loop(0, n_chunks // 2, body, 0)
    plsc.subcore_barrier()

    pltpu.sync_copy(acc_sh.at[pl.ds(s * ROWS_PER_TILE, ROWS_PER_TILE)],
                    out_hbm.at[c, pl.ds(s * ROWS_PER_TILE, ROWS_PER_TILE)])


_DN = (((1,), (1,)), ((), ()))
_BLK = 1000


def _tc_self_body(x_ref, ws_ref, b_ref, o_ref):
    o_ref[...] = lax.dot_general(x_ref[...], ws_ref[...], _DN,
                                 preferred_element_type=jnp.float32) + b_ref[...]


def _tc_self(feat, w_self, b2):
    return pl.pallas_call(
        _tc_self_body,
        grid=(N_NODES // _BLK,),
        in_specs=[
            pl.BlockSpec((_BLK, D_IN), lambda i: (i, 0)),
            pl.BlockSpec((D_OUT, D_IN), lambda i: (0, 0)),
            pl.BlockSpec((1, D_OUT), lambda i: (0, 0)),
        ],
        out_specs=pl.BlockSpec((_BLK, D_OUT), lambda i: (i, 0)),
        out_shape=jax.ShapeDtypeStruct((N_NODES, D_OUT), jnp.float32),
    )(feat, w_self, b2)


def _tc_combine_body(p0_ref, p1_ref, wn_ref, s_ref, o_ref):
    p = p0_ref[...] + p1_ref[...]
    h_neigh = p[:, :D_IN] / jnp.maximum(p[:, D_IN:D_IN + 1], 1.0)
    o_ref[...] = lax.dot_general(h_neigh, wn_ref[...], _DN,
                                 preferred_element_type=jnp.float32) + s_ref[...]


def _tc_combine(p0, p1, w_neigh, self_part):
    return pl.pallas_call(
        _tc_combine_body,
        grid=(N_NODES // _BLK,),
        in_specs=[
            pl.BlockSpec((_BLK, DE), lambda i: (i, 0)),
            pl.BlockSpec((_BLK, DE), lambda i: (i, 0)),
            pl.BlockSpec((D_OUT, D_IN), lambda i: (0, 0)),
            pl.BlockSpec((_BLK, D_OUT), lambda i: (i, 0)),
        ],
        out_specs=pl.BlockSpec((_BLK, D_OUT), lambda i: (i, 0)),
        out_shape=jax.ShapeDtypeStruct((N_NODES, D_OUT), jnp.float32),
    )(p0, p1, w_neigh, self_part)


def kernel(feat, edge_index, W_self, W_neigh, b_neigh):
    ones = jnp.ones((N_NODES, 1), jnp.float32)
    pad_cols = jnp.zeros((N_NODES, DE - D_IN - 1), jnp.float32)
    feat_ext = jnp.concatenate([feat, ones, pad_cols], axis=1)
    zeros_tile = jnp.zeros((ROWS_PER_TILE, DE), jnp.float32)

    b2 = b_neigh.reshape(1, D_OUT)
    self_part = _tc_self(feat, W_self, b2)
    partials = _sc_aggregate(feat_ext, edge_index, zeros_tile)
    return _tc_combine(partials[0], partials[1], W_neigh, self_part)


# speedup vs baseline: 3.4635x; 1.0067x over previous
import functools

import jax
import jax.numpy as jnp
from jax import lax
from jax.experimental import pallas as pl
from jax.experimental.pallas import tpu as pltpu
from jax.experimental.pallas import tpu_sc as plsc

N_NODES = 10000
D_IN = 128
D_OUT = 128
N_EDGES = 320000

DE = 144
NC = 2
NS = 16
NW = NC * NS
CHUNK = 128
N_CHUNKS = N_EDGES // CHUNK
CH_SC0 = 1248
CH_SC1 = N_CHUNKS - CH_SC0
CH1_BASE = CH_SC1 // NS
CH1_XTRA = CH_SC1 % NS
ACC_ROWS = 10240
ROWS_PER_TILE = ACC_ROWS // NS

_sc_mesh = plsc.VectorSubcoreMesh(
    core_axis_name="c", subcore_axis_name="s", num_cores=NC, num_subcores=NS)


@functools.partial(
    pl.kernel,
    out_type=jax.ShapeDtypeStruct((NC, ACC_ROWS, DE), jnp.float32),
    mesh=_sc_mesh,
    compiler_params=pltpu.CompilerParams(use_tc_tiling_on_sc=False),
    scratch_types=[
        pltpu.VMEM((CHUNK,), jnp.int32),
        pltpu.VMEM((CHUNK,), jnp.int32),
        pltpu.VMEM((CHUNK,), jnp.int32),
        pltpu.VMEM((CHUNK,), jnp.int32),
        pltpu.VMEM((CHUNK, DE), jnp.float32),
        pltpu.VMEM((CHUNK, DE), jnp.float32),
        pltpu.SemaphoreType.DMA,
        pltpu.SemaphoreType.DMA,
        pltpu.VMEM_SHARED((ACC_ROWS, DE), jnp.float32),
    ],
)
def _sc_aggregate(featext_hbm, edges_hbm, zeros_hbm, out_hbm,
                  src0, dst0, src1, dst1, rows0, rows1, gsem, isem, acc_sh):
    c = lax.axis_index("c")
    s = lax.axis_index("s")
    n_chunks = jnp.where(c == 0, CH_SC0 // NS, CH1_BASE + 2 * (s < CH1_XTRA // 2))
    chunk0 = jnp.where(c == 0, s * (CH_SC0 // NS),
                       CH_SC0 + s * CH1_BASE + 2 * jnp.minimum(s, CH1_XTRA // 2))

    def load_idx(k, sv, dv):
        base = (chunk0 + k) * CHUNK
        pltpu.async_copy(edges_hbm.at[0, pl.ds(base, CHUNK)], sv, isem)
        pltpu.async_copy(edges_hbm.at[1, pl.ds(base, CHUNK)], dv, isem)

    def wait_idx(sv, dv):
        pltpu.make_async_copy(edges_hbm.at[0, pl.ds(0, CHUNK)], sv, isem).wait()
        pltpu.make_async_copy(edges_hbm.at[1, pl.ds(0, CHUNK)], dv, isem).wait()

    def wait_rows(sv, buf):
        pltpu.make_async_copy(featext_hbm.at[sv], buf, gsem).wait()

    pltpu.sync_copy(zeros_hbm, acc_sh.at[pl.ds(s * ROWS_PER_TILE, ROWS_PER_TILE)])
    plsc.subcore_barrier()

    pltpu.sync_copy(edges_hbm.at[0, pl.ds(chunk0 * CHUNK, CHUNK)], src0)
    pltpu.sync_copy(edges_hbm.at[1, pl.ds(chunk0 * CHUNK, CHUNK)], dst0)
    pltpu.async_copy(featext_hbm.at[src0], rows0, gsem)
    load_idx(1, src1, dst1)

    def body(j2, carry):
        j = 2 * j2
        wait_idx(src1, dst1)
        wait_rows(src0, rows0)
        pltpu.async_copy(featext_hbm.at[src1], rows1, gsem)
        pltpu.sync_copy(rows0, acc_sh.at[dst0], add=True)

        @pl.when(j + 2 < n_chunks)
        def _():
            load_idx(j + 2, src0, dst0)

        wait_rows(src1, rows1)

        @pl.when(j + 2 < n_chunks)
        def _():
            wait_idx(src0, dst0)
            pltpu.async_copy(featext_hbm.at[src0], rows0, gsem)

        pltpu.sync_copy(rows1, acc_sh.at[dst1], add=True)

        @pl.when(j + 3 < n_chunks)
        def _():
            load_idx(j + 3, src1, dst1)

        return carry

    lax.fori_loop(0, n_chunks // 2, body, 0)
    plsc.subcore_barrier()

    pltpu.sync_copy(acc_sh.at[pl.ds(s * ROWS_PER_TILE, ROWS_PER_TILE)],
                    out_hbm.at[c, pl.ds(s * ROWS_PER_TILE, ROWS_PER_TILE)])


_DN = (((1,), (1,)), ((), ()))
_BLK = 2000


def _tc_self_body(x_ref, ws_ref, b_ref, o_ref):
    o_ref[...] = lax.dot_general(x_ref[...], ws_ref[...], _DN,
                                 preferred_element_type=jnp.float32) + b_ref[...]


def _tc_self(feat, w_self, b2):
    return pl.pallas_call(
        _tc_self_body,
        grid=(N_NODES // _BLK,),
        in_specs=[
            pl.BlockSpec((_BLK, D_IN), lambda i: (i, 0)),
            pl.BlockSpec((D_OUT, D_IN), lambda i: (0, 0)),
            pl.BlockSpec((1, D_OUT), lambda i: (0, 0)),
        ],
        out_specs=pl.BlockSpec((_BLK, D_OUT), lambda i: (i, 0)),
        out_shape=jax.ShapeDtypeStruct((N_NODES, D_OUT), jnp.float32),
    )(feat, w_self, b2)


def _tc_combine_body(p0_ref, p1_ref, wn_ref, s_ref, o_ref):
    p = p0_ref[...] + p1_ref[...]
    h_neigh = p[:, :D_IN] / jnp.maximum(p[:, D_IN:D_IN + 1], 1.0)
    o_ref[...] = lax.dot_general(h_neigh, wn_ref[...], _DN,
                                 preferred_element_type=jnp.float32) + s_ref[...]


def _tc_combine(p0, p1, w_neigh, self_part):
    return pl.pallas_call(
        _tc_combine_body,
        grid=(N_NODES // _BLK,),
        in_specs=[
            pl.BlockSpec((_BLK, DE), lambda i: (i, 0)),
            pl.BlockSpec((_BLK, DE), lambda i: (i, 0)),
            pl.BlockSpec((D_OUT, D_IN), lambda i: (0, 0)),
            pl.BlockSpec((_BLK, D_OUT), lambda i: (i, 0)),
        ],
        out_specs=pl.BlockSpec((_BLK, D_OUT), lambda i: (i, 0)),
        out_shape=jax.ShapeDtypeStruct((N_NODES, D_OUT), jnp.float32),
    )(p0, p1, w_neigh, self_part)


def kernel(feat, edge_index, W_self, W_neigh, b_neigh):
    ones_col = (jnp.arange(DE, dtype=jnp.int32) == D_IN).astype(jnp.float32)
    feat_ext = jnp.pad(feat, ((0, 0), (0, DE - D_IN))) + ones_col[None, :]
    zeros_tile = jnp.zeros((ROWS_PER_TILE, DE), jnp.float32)

    b2 = b_neigh.reshape(1, D_OUT)
    self_part = _tc_self(feat, W_self, b2)
    partials = _sc_aggregate(feat_ext, edge_index, zeros_tile)
    return _tc_combine(partials[0], partials[1], W_neigh, self_part)
